# baseline (speedup 1.0000x reference)
import jax
import jax.numpy as jnp
from jax.experimental import pallas as pl
from jax.experimental.pallas import tpu as pltpu

_SCALING = 32.0 / 16

_BM = 1024
_BN = 512

_PBN = 512


def _prep_body(w_ref, lb_ref, a_ref, weff_ref):
    upd = jax.lax.dot_general(
        lb_ref[...], a_ref[...], (((1,), (0,)), ((), ())),
        preferred_element_type=jnp.float32,
    )
    weff_ref[...] = (w_ref[...] + upd * _SCALING).astype(jnp.bfloat16)


def _serp(i, j):
    nj = 4096 // _BN
    return (jax.lax.select(i % 2 == 0, j, nj - 1 - j), 0)


def _x_copy(x_hbm, xbuf, sem, blk):
    return pltpu.make_async_copy(
        x_hbm.at[pl.ds(blk * _BM, _BM), :], xbuf.at[blk % 2], sem.at[blk % 2]
    )


def _mm_body(x_hbm, w_ref, b_ref, o_ref, xbuf, xbf, sem):
    i = pl.program_id(0)
    j = pl.program_id(1)
    ni = pl.num_programs(0)

    @pl.when((i == 0) & (j == 0))
    def _start_first():
        _x_copy(x_hbm, xbuf, sem, 0).start()

    @pl.when((j == 0) & (i + 1 < ni))
    def _prefetch_next():
        _x_copy(x_hbm, xbuf, sem, i + 1).start()

    @pl.when(j == 0)
    def _wait_current():
        _x_copy(x_hbm, xbuf, sem, i).wait()
        xbf[...] = xbuf[i % 2].astype(jnp.bfloat16)

    jj = _serp(i, j)[0]
    o_ref[...] = (
        jax.lax.dot_general(
            xbf[...], w_ref[...], (((1,), (1,)), ((), ())),
            preferred_element_type=jnp.float32,
        )
        + b_ref[:, pl.ds(jj * _BN, _BN)]
    )


def kernel(inputs, weight, bias, lora_a, lora_b):
    B, S, D_IN = inputs.shape
    D_OUT = weight.shape[0]
    R = lora_a.shape[0]
    M = B * S
    x2 = inputs.reshape(M, D_IN)
    b2 = bias.reshape(1, D_OUT)

    w_eff = pl.pallas_call(
        _prep_body,
        grid=(D_OUT // _PBN,),
        in_specs=[
            pl.BlockSpec((_PBN, D_IN), lambda j: (j, 0)),
            pl.BlockSpec((_PBN, R), lambda j: (j, 0)),
            pl.BlockSpec((R, D_IN), lambda j: (0, 0)),
        ],
        out_specs=pl.BlockSpec((_PBN, D_IN), lambda j: (j, 0)),
        out_shape=jax.ShapeDtypeStruct((D_OUT, D_IN), jnp.bfloat16),
        compiler_params=pltpu.CompilerParams(
            dimension_semantics=("arbitrary",),
        ),
    )(weight, lora_b, lora_a)

    out = pl.pallas_call(
        _mm_body,
        grid=(M // _BM, D_OUT // _BN),
        in_specs=[
            pl.BlockSpec(memory_space=pl.ANY),
            pl.BlockSpec((_BN, D_IN), _serp),
            pl.BlockSpec((1, 4096), lambda i, j: (0, 0)),
        ],
        out_specs=pl.BlockSpec((_BM, _BN), lambda i, j: (i, _serp(i, j)[0])),
        out_shape=jax.ShapeDtypeStruct((M, D_OUT), jnp.float32),
        scratch_shapes=[
            pltpu.VMEM((2, _BM, D_IN), jnp.float32),
            pltpu.VMEM((_BM, D_IN), jnp.bfloat16),
            pltpu.SemaphoreType.DMA((2,)),
        ],
        compiler_params=pltpu.CompilerParams(
            dimension_semantics=("parallel", "arbitrary"),
        ),
    )(x2, w_eff, b2)
    return out.reshape(B, S, D_OUT)


# speedup vs baseline: 1.0534x; 1.0534x over previous
import jax
import jax.numpy as jnp
from jax.experimental import pallas as pl
from jax.experimental.pallas import tpu as pltpu

_SCALING = 32.0 / 16

_BM = 1024
_BN = 512
_HBN = 512


def _serp(i, j):
    nj = 4096 // _BN
    return (jax.lax.select(i % 2 == 0, j, nj - 1 - j), 0)


def _head_body(x_ref, w_ref, lb_ref, a_ref, b_ref, o_ref, weff_ref):
    j = pl.program_id(0)
    upd = jax.lax.dot_general(
        lb_ref[...], a_ref[...], (((1,), (0,)), ((), ())),
        preferred_element_type=jnp.float32,
    )
    weff = (w_ref[...] + upd * _SCALING).astype(jnp.bfloat16)
    weff_ref[...] = weff
    xb = x_ref[...].astype(jnp.bfloat16)
    o_ref[...] = (
        jax.lax.dot_general(
            xb, weff, (((1,), (1,)), ((), ())),
            preferred_element_type=jnp.float32,
        )
        + b_ref[:, pl.ds(j * _HBN, _HBN)]
    )


def _x_copy(x_hbm, xbuf, sem, blk):
    return pltpu.make_async_copy(
        x_hbm.at[pl.ds((blk + 1) * _BM, _BM), :], xbuf.at[blk % 2], sem.at[blk % 2]
    )


def _tail_body(x_hbm, w_ref, b_ref, o_prev, o_ref, xbuf, sem):
    i = pl.program_id(0)
    j = pl.program_id(1)
    ni = pl.num_programs(0)

    @pl.when((i == 0) & (j == 0))
    def _start_first():
        _x_copy(x_hbm, xbuf, sem, 0).start()

    @pl.when((j == 0) & (i + 1 < ni))
    def _prefetch_next():
        _x_copy(x_hbm, xbuf, sem, i + 1).start()

    @pl.when(j == 0)
    def _wait_current():
        _x_copy(x_hbm, xbuf, sem, i).wait()

    jj = _serp(i, j)[0]
    xb = xbuf[i % 2].astype(jnp.bfloat16)
    o_ref[...] = (
        jax.lax.dot_general(
            xb, w_ref[...], (((1,), (1,)), ((), ())),
            preferred_element_type=jnp.float32,
        )
        + b_ref[:, pl.ds(jj * _BN, _BN)]
    )


def kernel(inputs, weight, bias, lora_a, lora_b):
    B, S, D_IN = inputs.shape
    D_OUT = weight.shape[0]
    R = lora_a.shape[0]
    M = B * S
    x2 = inputs.reshape(M, D_IN)
    b2 = bias.reshape(1, D_OUT)

    out_head, w_eff = pl.pallas_call(
        _head_body,
        grid=(D_OUT // _HBN,),
        in_specs=[
            pl.BlockSpec(
                (_BM, D_IN), lambda j: (0, 0),
                pipeline_mode=pl.Buffered(buffer_count=1),
            ),
            pl.BlockSpec((_HBN, D_IN), lambda j: (j, 0)),
            pl.BlockSpec((_HBN, R), lambda j: (j, 0)),
            pl.BlockSpec((R, D_IN), lambda j: (0, 0)),
            pl.BlockSpec((1, D_OUT), lambda j: (0, 0)),
        ],
        out_specs=[
            pl.BlockSpec((_BM, _HBN), lambda j: (0, j)),
            pl.BlockSpec((_HBN, D_IN), lambda j: (j, 0)),
        ],
        out_shape=[
            jax.ShapeDtypeStruct((M, D_OUT), jnp.float32),
            jax.ShapeDtypeStruct((D_OUT, D_IN), jnp.bfloat16),
        ],
        compiler_params=pltpu.CompilerParams(
            dimension_semantics=("arbitrary",),
        ),
    )(x2, weight, lora_b, lora_a, b2)

    out = pl.pallas_call(
        _tail_body,
        grid=((M - _BM) // _BM, D_OUT // _BN),
        in_specs=[
            pl.BlockSpec(memory_space=pl.ANY),
            pl.BlockSpec((_BN, D_IN), _serp),
            pl.BlockSpec((1, D_OUT), lambda i, j: (0, 0)),
            pl.BlockSpec(memory_space=pl.ANY),
        ],
        out_specs=pl.BlockSpec((_BM, _BN), lambda i, j: (i + 1, _serp(i, j)[0])),
        out_shape=jax.ShapeDtypeStruct((M, D_OUT), jnp.float32),
        input_output_aliases={3: 0},
        scratch_shapes=[
            pltpu.VMEM((2, _BM, D_IN), jnp.float32),
            pltpu.SemaphoreType.DMA((2,)),
        ],
        compiler_params=pltpu.CompilerParams(
            dimension_semantics=("parallel", "arbitrary"),
        ),
    )(x2, w_eff, b2, out_head)
    return out.reshape(B, S, D_OUT)
